# baseline (speedup 1.0000x reference)
import functools

import jax
import jax.numpy as jnp
from jax.experimental import pallas as pl

K = 32
BLK = 4096


def _sae_block(x_ref, we_ref, be_ref, wd_ref, bd_ref, xhat_ref, zs_ref):
    x = x_ref[...]
    z = jnp.maximum(jnp.dot(x, we_ref[...], preferred_element_type=jnp.float32)
                    + be_ref[...], 0.0)
    zb = jnp.maximum(z.view(jnp.int32), 0)
    lo = jnp.zeros((z.shape[0], 1), jnp.int32)

    def body(i, lo):
        trial = lo | (jnp.int32(1) << (30 - i))
        cnt = jnp.sum(zb >= trial, axis=1, keepdims=True, dtype=jnp.float32)
        return jnp.where(cnt >= float(K), trial, lo)

    lo = jax.lax.fori_loop(0, 31, body, lo, unroll=True)
    thr = lo.view(jnp.float32)
    zs = jnp.where(z >= thr, z, 0.0)
    zs_ref[...] = zs
    xhat_ref[...] = jnp.dot(zs, wd_ref[...], preferred_element_type=jnp.float32) + bd_ref[...]


@jax.jit
def kernel(x, W_enc, b_enc, W_dec, b_dec):
    B, D_IN = x.shape
    D_LAT = W_enc.shape[1]
    grid = (B // BLK,)
    xhat, zs = pl.pallas_call(
        _sae_block,
        grid=grid,
        in_specs=[
            pl.BlockSpec((BLK, D_IN), lambda i: (i, 0)),
            pl.BlockSpec((D_IN, D_LAT), lambda i: (0, 0)),
            pl.BlockSpec((1, D_LAT), lambda i: (0, 0)),
            pl.BlockSpec((D_LAT, D_IN), lambda i: (0, 0)),
            pl.BlockSpec((1, D_IN), lambda i: (0, 0)),
        ],
        out_specs=[
            pl.BlockSpec((BLK, D_IN), lambda i: (i, 0)),
            pl.BlockSpec((BLK, D_LAT), lambda i: (i, 0)),
        ],
        out_shape=[
            jax.ShapeDtypeStruct((B, D_IN), jnp.float32),
            jax.ShapeDtypeStruct((B, D_LAT), jnp.float32),
        ],
    )(x, W_enc, b_enc.reshape(1, D_LAT), W_dec, b_dec.reshape(1, D_IN))
    return xhat, zs


# speedup vs baseline: 1.0016x; 1.0016x over previous
import functools

import jax
import jax.numpy as jnp
from jax.experimental import pallas as pl

K = 32
BLK = 2048


def _sae_block(x_ref, we_ref, be_ref, wd_ref, bd_ref, xhat_ref, zs_ref):
    x = x_ref[...]
    z = jnp.maximum(jnp.dot(x, we_ref[...], preferred_element_type=jnp.float32)
                    + be_ref[...], 0.0)
    zb = jnp.maximum(z.view(jnp.int32), 0)
    lo = jnp.zeros((z.shape[0], 1), jnp.int32)

    def body(i, lo):
        trial = lo | (jnp.int32(1) << (30 - i))
        cnt = jnp.sum(zb >= trial, axis=1, keepdims=True, dtype=jnp.float32)
        return jnp.where(cnt >= float(K), trial, lo)

    lo = jax.lax.fori_loop(0, 31, body, lo, unroll=True)
    thr = lo.view(jnp.float32)
    zs = jnp.where(z >= thr, z, 0.0)
    zs_ref[...] = zs
    xhat_ref[...] = jnp.dot(zs, wd_ref[...], preferred_element_type=jnp.float32) + bd_ref[...]


@jax.jit
def kernel(x, W_enc, b_enc, W_dec, b_dec):
    B, D_IN = x.shape
    D_LAT = W_enc.shape[1]
    grid = (B // BLK,)
    xhat, zs = pl.pallas_call(
        _sae_block,
        grid=grid,
        in_specs=[
            pl.BlockSpec((BLK, D_IN), lambda i: (i, 0)),
            pl.BlockSpec((D_IN, D_LAT), lambda i: (0, 0)),
            pl.BlockSpec((1, D_LAT), lambda i: (0, 0)),
            pl.BlockSpec((D_LAT, D_IN), lambda i: (0, 0)),
            pl.BlockSpec((1, D_IN), lambda i: (0, 0)),
        ],
        out_specs=[
            pl.BlockSpec((BLK, D_IN), lambda i: (i, 0)),
            pl.BlockSpec((BLK, D_LAT), lambda i: (i, 0)),
        ],
        out_shape=[
            jax.ShapeDtypeStruct((B, D_IN), jnp.float32),
            jax.ShapeDtypeStruct((B, D_LAT), jnp.float32),
        ],
    )(x, W_enc, b_enc.reshape(1, D_LAT), W_dec, b_dec.reshape(1, D_IN))
    return xhat, zs


# speedup vs baseline: 1.0162x; 1.0146x over previous
import functools

import jax
import jax.numpy as jnp
from jax.experimental import pallas as pl

K = 32
BLK = 2048


def _sae_block(x_ref, we_ref, be_ref, wd_ref, bd_ref, xhat_ref, zs_ref):
    x = x_ref[...]
    z = jnp.maximum(jnp.dot(x, we_ref[...], preferred_element_type=jnp.float32)
                    + be_ref[...], 0.0)
    lo = jnp.zeros((z.shape[0], 1), jnp.int32)

    def body(i, lo):
        trial = lo | (jnp.int32(1) << (30 - i))
        cnt = jnp.sum(z >= trial.view(jnp.float32), axis=1, keepdims=True,
                      dtype=jnp.float32)
        return jnp.where(cnt >= float(K), trial, lo)

    lo = jax.lax.fori_loop(0, 31, body, lo, unroll=True)
    thr = lo.view(jnp.float32)
    zs = jnp.where(z >= thr, z, 0.0)
    zs_ref[...] = zs
    xhat_ref[...] = jnp.dot(zs, wd_ref[...], preferred_element_type=jnp.float32) + bd_ref[...]


@jax.jit
def kernel(x, W_enc, b_enc, W_dec, b_dec):
    B, D_IN = x.shape
    D_LAT = W_enc.shape[1]
    grid = (B // BLK,)
    xhat, zs = pl.pallas_call(
        _sae_block,
        grid=grid,
        in_specs=[
            pl.BlockSpec((BLK, D_IN), lambda i: (i, 0)),
            pl.BlockSpec((D_IN, D_LAT), lambda i: (0, 0)),
            pl.BlockSpec((1, D_LAT), lambda i: (0, 0)),
            pl.BlockSpec((D_LAT, D_IN), lambda i: (0, 0)),
            pl.BlockSpec((1, D_IN), lambda i: (0, 0)),
        ],
        out_specs=[
            pl.BlockSpec((BLK, D_IN), lambda i: (i, 0)),
            pl.BlockSpec((BLK, D_LAT), lambda i: (i, 0)),
        ],
        out_shape=[
            jax.ShapeDtypeStruct((B, D_IN), jnp.float32),
            jax.ShapeDtypeStruct((B, D_LAT), jnp.float32),
        ],
    )(x, W_enc, b_enc.reshape(1, D_LAT), W_dec, b_dec.reshape(1, D_IN))
    return xhat, zs
